# baseline (speedup 1.0000x reference)
import functools

import jax
import jax.numpy as jnp
from jax import lax
from jax.experimental import pallas as pl
from jax.experimental.pallas import tpu as pltpu
from jax.experimental.pallas import tpu_sc as plsc

_V = 100001
_D = 32
_LANES = 16
_CT = 128
_NFULL_TILES = _V // _CT
_TAIL = _V - _NFULL_TILES * _CT
_NW = 32
_K_PER_W = _NFULL_TILES // _NW
_EXTRA0 = _NW * _K_PER_W
_N_EXTRA = _NFULL_TILES - _EXTRA0
_TAIL_GRP = 3


def _make_detile():
    mesh = plsc.VectorSubcoreMesh(core_axis_name="c", subcore_axis_name="s")

    @functools.partial(
        pl.kernel,
        mesh=mesh,
        out_type=jax.ShapeDtypeStruct((_V * _D,), jnp.float32),
        scratch_types=[
            pltpu.VMEM((_K_PER_W, _D, _CT), jnp.float32),
            pltpu.VMEM((_D, _CT), jnp.float32),
            pltpu.VMEM((_D, _CT), jnp.float32),
            pltpu.VMEM((_CT * _D,), jnp.float32),
            pltpu.VMEM((_CT * _D,), jnp.float32),
            pltpu.SemaphoreType.DMA,
            pltpu.SemaphoreType.DMA,
            pltpu.SemaphoreType.DMA,
        ],
        compiler_params=pltpu.CompilerParams(
            use_tc_tiling_on_sc=True, needs_layout_passes=False,
            disable_bounds_checks=True),
    )
    def detile_kernel(tt_hbm, out_hbm, chunks_v, chunk_x, chunk_t,
                      trans_a, trans_b, sem_in, sem_a, sem_b):
        wid = lax.axis_index("s") * 2 + lax.axis_index("c")
        base_ct = wid * _K_PER_W

        in_handles = []
        for k in range(_K_PER_W):
            h = pltpu.make_async_copy(
                tt_hbm.at[:, pl.ds((base_ct + k) * _CT, _CT)],
                chunks_v.at[k], sem_in)
            h.start()
            in_handles.append(h)
        hx = pltpu.make_async_copy(
            tt_hbm.at[:, pl.ds((_EXTRA0 + wid) * _CT, _CT)], chunk_x, sem_in)
        tail_off = pl.multiple_of(
            jnp.int32(_NFULL_TILES * _CT) + wid * 0, _CT)
        ht = pltpu.make_async_copy(
            tt_hbm.at[:, pl.ds(tail_off, _CT)], chunk_t, sem_in)
        pl.when(wid < _N_EXTRA)(hx.start)
        pl.when(wid == _N_EXTRA)(ht.start)
        for h in in_handles:
            h.wait()
        pl.when(wid < _N_EXTRA)(hx.wait)
        pl.when(wid == _N_EXTRA)(ht.wait)

        iota32 = lax.iota(jnp.int32, _LANES) * _D
        ngrp = _CT // _LANES

        def transpose_chunk(src, trans, width_grp):
            def body(cg, _):
                base = cg * (_LANES * _D)
                for d in range(_D):
                    v = src[d, pl.ds(cg * _LANES, _LANES)]
                    idx = iota32 + (base + d)
                    plsc.store_scatter(trans, [idx], v)
                return 0
            lax.fori_loop(0, width_grp, body, 0, unroll=False)

        def out_copy(trans, off, n, sem):
            h = pltpu.make_async_copy(
                trans.at[pl.ds(0, n)], out_hbm.at[pl.ds(off, n)], sem)
            h.start()
            return h

        pending = [None, None]
        slots = [(trans_a, sem_a), (trans_b, sem_b)]

        for k in range(_K_PER_W):
            s = k % 2
            trans, sem = slots[s]
            if pending[s] is not None:
                pending[s].wait()
            transpose_chunk(chunks_v.at[k], trans, ngrp)
            pending[s] = out_copy(trans, (base_ct + k) * _CT * _D,
                                  _CT * _D, sem)

        for s in range(2):
            if pending[s] is not None:
                pending[s].wait()
                pending[s] = None
        trans, sem = slots[0]

        @pl.when(wid < _N_EXTRA)
        def _():
            transpose_chunk(chunk_x, trans, ngrp)
            out_copy(trans, (_EXTRA0 + wid) * _CT * _D, _CT * _D, sem).wait()

        @pl.when(wid == _N_EXTRA)
        def _():
            transpose_chunk(chunk_t, trans, _TAIL_GRP)
            out_copy(trans, _NFULL_TILES * _CT * _D, _TAIL * _D, sem).wait()

    return detile_kernel


def _make_gather(batch: int, dim: int):
    info = plsc.get_sparse_core_info()
    num_workers = info.num_cores * info.num_subcores
    b_per_w = batch // num_workers
    mesh = plsc.VectorSubcoreMesh(core_axis_name="c", subcore_axis_name="s")

    @functools.partial(
        pl.kernel,
        mesh=mesh,
        out_type=jax.ShapeDtypeStruct((batch, dim), jnp.float32),
        scratch_types=[
            pltpu.VMEM((b_per_w,), jnp.int32),
            pltpu.VMEM((b_per_w, dim), jnp.float32),
            pltpu.SemaphoreType.DMA,
        ],
        compiler_params=pltpu.CompilerParams(use_tc_tiling_on_sc=False),
    )
    def gather_kernel(table_hbm, idx_hbm, out_hbm, idx_v, rows_v, sem):
        wid = lax.axis_index("s") * info.num_cores + lax.axis_index("c")
        base = wid * b_per_w
        pltpu.sync_copy(idx_hbm.at[pl.ds(base, b_per_w)], idx_v)
        pltpu.async_copy(table_hbm.at[idx_v], rows_v, sem).wait()
        pltpu.sync_copy(rows_v, out_hbm.at[pl.ds(base, b_per_w)])

    return gather_kernel


@functools.lru_cache(maxsize=None)
def _pipeline(batch, dim):
    detile = _make_detile()
    gather = _make_gather(batch, dim)

    def run(indices, table):
        tlin = detile(table.T).reshape(_V, _D)
        return gather(tlin, indices.astype(jnp.int32))

    return run


def kernel(indices, table):
    batch, = indices.shape
    _, dim = table.shape
    return _pipeline(batch, dim)(indices, table)


# speedup vs baseline: 1.0661x; 1.0661x over previous
import functools

import jax
import jax.numpy as jnp
from jax import lax
from jax.experimental import pallas as pl
from jax.experimental.pallas import tpu as pltpu
from jax.experimental.pallas import tpu_sc as plsc

_V = 100001
_D = 32
_LANES = 16
_CT = 128
_NFULL_TILES = _V // _CT
_TAIL = _V - _NFULL_TILES * _CT
_NW = 32
_K_PER_W = _NFULL_TILES // _NW
_EXTRA0 = _NW * _K_PER_W
_N_EXTRA = _NFULL_TILES - _EXTRA0
_TAIL_GRP = 3


def _make_detile():
    mesh = plsc.VectorSubcoreMesh(core_axis_name="c", subcore_axis_name="s")

    @functools.partial(
        pl.kernel,
        mesh=mesh,
        out_type=jax.ShapeDtypeStruct((_V * _D,), jnp.float32),
        scratch_types=[
            pltpu.VMEM((_K_PER_W, _D, _CT), jnp.float32),
            pltpu.VMEM((_D, _CT), jnp.float32),
            pltpu.VMEM((_D, _CT), jnp.float32),
            pltpu.VMEM((_CT * _D,), jnp.float32),
            pltpu.VMEM((_CT * _D,), jnp.float32),
            pltpu.SemaphoreType.DMA((_K_PER_W,)),
            pltpu.SemaphoreType.DMA,
            pltpu.SemaphoreType.DMA,
            pltpu.SemaphoreType.DMA,
        ],
        compiler_params=pltpu.CompilerParams(
            use_tc_tiling_on_sc=True, needs_layout_passes=False,
            disable_bounds_checks=True),
    )
    def detile_kernel(tt_hbm, out_hbm, chunks_v, chunk_x, chunk_t,
                      trans_a, trans_b, sems_in, sem_x, sem_a, sem_b):
        wid = lax.axis_index("s") * 2 + lax.axis_index("c")
        base_ct = wid * _K_PER_W

        in_handles = []
        for k in range(_K_PER_W):
            h = pltpu.make_async_copy(
                tt_hbm.at[:, pl.ds((base_ct + k) * _CT, _CT)],
                chunks_v.at[k], sems_in.at[k])
            h.start()
            in_handles.append(h)
        hx = pltpu.make_async_copy(
            tt_hbm.at[:, pl.ds((_EXTRA0 + wid) * _CT, _CT)], chunk_x, sem_x)
        tail_off = pl.multiple_of(
            jnp.int32(_NFULL_TILES * _CT) + wid * 0, _CT)
        ht = pltpu.make_async_copy(
            tt_hbm.at[:, pl.ds(tail_off, _CT)], chunk_t, sem_x)
        pl.when(wid < _N_EXTRA)(hx.start)
        pl.when(wid == _N_EXTRA)(ht.start)

        iota32 = lax.iota(jnp.int32, _LANES) * _D
        ngrp = _CT // _LANES
        idx_bases = [iota32 + cg * (_LANES * _D) for cg in range(ngrp)]

        def transpose_chunk(src, trans):
            def body(d, _):
                for cg in range(ngrp):
                    v = src[d, pl.ds(cg * _LANES, _LANES)]
                    plsc.store_scatter(trans, [idx_bases[cg] + d], v)
                return 0
            lax.fori_loop(0, _D, body, 0, unroll=False)

        def out_copy(trans, off, n, sem):
            h = pltpu.make_async_copy(
                trans.at[pl.ds(0, n)], out_hbm.at[pl.ds(off, n)], sem)
            h.start()
            return h

        pending = [None, None]
        slots = [(trans_a, sem_a), (trans_b, sem_b)]

        for k in range(_K_PER_W):
            s = k % 2
            trans, sem = slots[s]
            in_handles[k].wait()
            if pending[s] is not None:
                pending[s].wait()
            transpose_chunk(chunks_v.at[k], trans)
            pending[s] = out_copy(trans, (base_ct + k) * _CT * _D,
                                  _CT * _D, sem)

        for s in range(2):
            if pending[s] is not None:
                pending[s].wait()
                pending[s] = None
        trans, sem = slots[0]

        @pl.when(wid < _N_EXTRA)
        def _():
            hx.wait()
            transpose_chunk(chunk_x, trans)
            out_copy(trans, (_EXTRA0 + wid) * _CT * _D, _CT * _D, sem).wait()

        @pl.when(wid == _N_EXTRA)
        def _():
            ht.wait()
            transpose_chunk(chunk_t, trans)
            out_copy(trans, _NFULL_TILES * _CT * _D, _TAIL * _D, sem).wait()

    return detile_kernel


def _make_gather(batch: int, dim: int):
    info = plsc.get_sparse_core_info()
    num_workers = info.num_cores * info.num_subcores
    b_per_w = batch // num_workers
    mesh = plsc.VectorSubcoreMesh(core_axis_name="c", subcore_axis_name="s")

    @functools.partial(
        pl.kernel,
        mesh=mesh,
        out_type=jax.ShapeDtypeStruct((batch, dim), jnp.float32),
        scratch_types=[
            pltpu.VMEM((b_per_w,), jnp.int32),
            pltpu.VMEM((b_per_w, dim), jnp.float32),
            pltpu.SemaphoreType.DMA,
        ],
        compiler_params=pltpu.CompilerParams(use_tc_tiling_on_sc=False),
    )
    def gather_kernel(table_hbm, idx_hbm, out_hbm, idx_v, rows_v, sem):
        wid = lax.axis_index("s") * info.num_cores + lax.axis_index("c")
        base = wid * b_per_w
        pltpu.sync_copy(idx_hbm.at[pl.ds(base, b_per_w)], idx_v)
        pltpu.async_copy(table_hbm.at[idx_v], rows_v, sem).wait()
        pltpu.sync_copy(rows_v, out_hbm.at[pl.ds(base, b_per_w)])

    return gather_kernel


@functools.lru_cache(maxsize=None)
def _pipeline(batch, dim):
    detile = _make_detile()
    gather = _make_gather(batch, dim)

    def run(indices, table):
        tlin = detile(table.T).reshape(_V, _D)
        return gather(tlin, indices.astype(jnp.int32))

    return run


def kernel(indices, table):
    batch, = indices.shape
    _, dim = table.shape
    return _pipeline(batch, dim)(indices, table)


# speedup vs baseline: 1.6347x; 1.5333x over previous
import functools

import jax
import jax.numpy as jnp
from jax import lax
from jax.experimental import pallas as pl
from jax.experimental.pallas import tpu as pltpu
from jax.experimental.pallas import tpu_sc as plsc

_V = 100001
_D = 32
_LANES = 16
_CT = 128
_NFULL_TILES = _V // _CT
_TAIL = _V - _NFULL_TILES * _CT
_NW = 32
_K_PER_W = _NFULL_TILES // _NW
_EXTRA0 = _NW * _K_PER_W
_N_EXTRA = _NFULL_TILES - _EXTRA0
_TAIL_GRP = 3


def _make_detile():
    mesh = plsc.VectorSubcoreMesh(core_axis_name="c", subcore_axis_name="s")

    @functools.partial(
        pl.kernel,
        mesh=mesh,
        out_type=jax.ShapeDtypeStruct((_V * _D,), jnp.float32),
        scratch_types=[
            pltpu.VMEM((_K_PER_W, _D, _CT), jnp.float32),
            pltpu.VMEM((_D, _CT), jnp.float32),
            pltpu.VMEM((_D, _CT), jnp.float32),
            pltpu.VMEM((_CT * _D,), jnp.float32),
            pltpu.VMEM((_CT * _D,), jnp.float32),
            pltpu.SemaphoreType.DMA((_K_PER_W,)),
            pltpu.SemaphoreType.DMA,
            pltpu.SemaphoreType.DMA,
            pltpu.SemaphoreType.DMA,
        ],
        compiler_params=pltpu.CompilerParams(
            use_tc_tiling_on_sc=True, needs_layout_passes=False,
            disable_bounds_checks=True),
    )
    def detile_kernel(tt_hbm, out_hbm, chunks_v, chunk_x, chunk_t,
                      trans_a, trans_b, sems_in, sem_x, sem_a, sem_b):
        wid = lax.axis_index("s") * 2 + lax.axis_index("c")
        base_ct = wid * _K_PER_W

        in_handles = []
        for k in range(_K_PER_W):
            h = pltpu.make_async_copy(
                tt_hbm.at[:, pl.ds((base_ct + k) * _CT, _CT)],
                chunks_v.at[k], sems_in.at[k])
            h.start()
            in_handles.append(h)
        hx = pltpu.make_async_copy(
            tt_hbm.at[:, pl.ds((_EXTRA0 + wid) * _CT, _CT)], chunk_x, sem_x)
        tail_off = pl.multiple_of(
            jnp.int32(_NFULL_TILES * _CT) + wid * 0, _CT)
        ht = pltpu.make_async_copy(
            tt_hbm.at[:, pl.ds(tail_off, _CT)], chunk_t, sem_x)
        pl.when(wid < _N_EXTRA)(hx.start)
        pl.when(wid == _N_EXTRA)(ht.start)

        iota = lax.iota(jnp.int32, _LANES)
        ngrp = _CT // _LANES
        rot = [(iota + r) & (_LANES - 1) for r in range(_LANES)]
        trix = [rot[r] * _D + iota for r in range(_LANES)]

        def transpose_chunk(src, trans):
            def body(g, _):
                bj = g * _LANES
                for bd in (0, _LANES):
                    dvec = iota + bd
                    base_tr = bj * _D + bd
                    for r in range(_LANES):
                        v = plsc.load_gather(src, [dvec, rot[r] + bj])
                        plsc.store_scatter(trans, [trix[r] + base_tr], v)
                return 0
            lax.fori_loop(0, ngrp, body, 0, unroll=False)

        def out_copy(trans, off, n, sem):
            h = pltpu.make_async_copy(
                trans.at[pl.ds(0, n)], out_hbm.at[pl.ds(off, n)], sem)
            h.start()
            return h

        def drain(sem):
            pltpu.make_async_copy(
                out_hbm.at[pl.ds(0, _CT * _D)], trans_a, sem).wait()

        def wait_in(k):
            pltpu.make_async_copy(
                tt_hbm.at[:, pl.ds(0, _CT)], chunks_v.at[k],
                sems_in.at[k]).wait()

        def chunk_body(k2, _):
            k = k2 * 2
            for s, (trans, sem) in enumerate(((trans_a, sem_a),
                                              (trans_b, sem_b))):
                pl.when(k2 > 0)(lambda sem=sem: drain(sem))
                wait_in(k + s)
                transpose_chunk(chunks_v.at[k + s], trans)
                out_copy(trans, (base_ct + k + s) * _CT * _D, _CT * _D, sem)
            return 0

        lax.fori_loop(0, _K_PER_W // 2, chunk_body, 0, unroll=False)
        drain(sem_a)
        drain(sem_b)
        trans, sem = trans_a, sem_a

        @pl.when(wid < _N_EXTRA)
        def _():
            hx.wait()
            transpose_chunk(chunk_x, trans)
            out_copy(trans, (_EXTRA0 + wid) * _CT * _D, _CT * _D, sem).wait()

        @pl.when(wid == _N_EXTRA)
        def _():
            ht.wait()
            transpose_chunk(chunk_t, trans)
            out_copy(trans, _NFULL_TILES * _CT * _D, _TAIL * _D, sem).wait()

    return detile_kernel


def _make_gather(batch: int, dim: int):
    info = plsc.get_sparse_core_info()
    num_workers = info.num_cores * info.num_subcores
    b_per_w = batch // num_workers
    mesh = plsc.VectorSubcoreMesh(core_axis_name="c", subcore_axis_name="s")

    @functools.partial(
        pl.kernel,
        mesh=mesh,
        out_type=jax.ShapeDtypeStruct((batch, dim), jnp.float32),
        scratch_types=[
            pltpu.VMEM((b_per_w,), jnp.int32),
            pltpu.VMEM((b_per_w, dim), jnp.float32),
            pltpu.SemaphoreType.DMA,
        ],
        compiler_params=pltpu.CompilerParams(use_tc_tiling_on_sc=False),
    )
    def gather_kernel(table_hbm, idx_hbm, out_hbm, idx_v, rows_v, sem):
        wid = lax.axis_index("s") * info.num_cores + lax.axis_index("c")
        base = wid * b_per_w
        pltpu.sync_copy(idx_hbm.at[pl.ds(base, b_per_w)], idx_v)
        pltpu.async_copy(table_hbm.at[idx_v], rows_v, sem).wait()
        pltpu.sync_copy(rows_v, out_hbm.at[pl.ds(base, b_per_w)])

    return gather_kernel


@functools.lru_cache(maxsize=None)
def _pipeline(batch, dim):
    detile = _make_detile()
    gather = _make_gather(batch, dim)

    def run(indices, table):
        tlin = detile(table.T).reshape(_V, _D)
        return gather(tlin, indices.astype(jnp.int32))

    return run


def kernel(indices, table):
    batch, = indices.shape
    _, dim = table.shape
    return _pipeline(batch, dim)(indices, table)


# speedup vs baseline: 1.7428x; 1.0661x over previous
import functools

import jax
import jax.numpy as jnp
from jax import lax
from jax.experimental import pallas as pl
from jax.experimental.pallas import tpu as pltpu
from jax.experimental.pallas import tpu_sc as plsc

_V = 100001
_D = 32
_LANES = 16
_CT = 128
_NFULL_TILES = _V // _CT
_TAIL = _V - _NFULL_TILES * _CT
_NW = 32
_K_PER_W = _NFULL_TILES // _NW
_EXTRA0 = _NW * _K_PER_W
_N_EXTRA = _NFULL_TILES - _EXTRA0
_TAIL_GRP = 3


def _make_detile():
    mesh = plsc.VectorSubcoreMesh(core_axis_name="c", subcore_axis_name="s")

    @functools.partial(
        pl.kernel,
        mesh=mesh,
        out_type=jax.ShapeDtypeStruct((_V * _D,), jnp.float32),
        scratch_types=[
            pltpu.VMEM((_K_PER_W, _D, _CT), jnp.float32),
            pltpu.VMEM((_D, _CT), jnp.float32),
            pltpu.VMEM((_D, _CT), jnp.float32),
            pltpu.VMEM((_CT * _D,), jnp.float32),
            pltpu.VMEM((_CT * _D,), jnp.float32),
            pltpu.VMEM((_CT * _D,), jnp.float32),
            pltpu.VMEM((_CT * _D,), jnp.float32),
            pltpu.SemaphoreType.DMA((_K_PER_W,)),
            pltpu.SemaphoreType.DMA,
            pltpu.SemaphoreType.DMA,
            pltpu.SemaphoreType.DMA,
            pltpu.SemaphoreType.DMA,
            pltpu.SemaphoreType.DMA,
        ],
        compiler_params=pltpu.CompilerParams(
            use_tc_tiling_on_sc=True, needs_layout_passes=False,
            disable_bounds_checks=True),
    )
    def detile_kernel(tt_hbm, out_hbm, chunks_v, chunk_x, chunk_t,
                      trans_a, trans_b, trans_c, trans_d, sems_in, sem_x,
                      sem_a, sem_b, sem_c, sem_d):
        wid = lax.axis_index("s") * 2 + lax.axis_index("c")
        base_ct = wid * _K_PER_W

        in_handles = []
        for k in range(_K_PER_W):
            h = pltpu.make_async_copy(
                tt_hbm.at[:, pl.ds((base_ct + k) * _CT, _CT)],
                chunks_v.at[k], sems_in.at[k])
            h.start()
            in_handles.append(h)
        hx = pltpu.make_async_copy(
            tt_hbm.at[:, pl.ds((_EXTRA0 + wid) * _CT, _CT)], chunk_x, sem_x)
        tail_off = pl.multiple_of(
            jnp.int32(_NFULL_TILES * _CT) + wid * 0, _CT)
        ht = pltpu.make_async_copy(
            tt_hbm.at[:, pl.ds(tail_off, _CT)], chunk_t, sem_x)
        pl.when(wid < _N_EXTRA)(hx.start)
        pl.when(wid == _N_EXTRA)(ht.start)

        iota = lax.iota(jnp.int32, _LANES)
        ngrp = _CT // _LANES
        rot = [(iota + r) & (_LANES - 1) for r in range(_LANES)]
        trix = [rot[r] * _D + iota for r in range(_LANES)]

        def transpose_chunk(src, trans):
            def body(g, _):
                bj = g * _LANES
                for bd in (0, _LANES):
                    dvec = iota + bd
                    base_tr = bj * _D + bd
                    for r in range(_LANES):
                        v = plsc.load_gather(src, [dvec, rot[r] + bj])
                        plsc.store_scatter(trans, [trix[r] + base_tr], v)
                return 0
            lax.fori_loop(0, ngrp, body, 0, unroll=False)

        def out_copy(trans, off, n, sem):
            h = pltpu.make_async_copy(
                trans.at[pl.ds(0, n)], out_hbm.at[pl.ds(off, n)], sem)
            h.start()
            return h

        def drain(sem):
            pltpu.make_async_copy(
                out_hbm.at[pl.ds(0, _CT * _D)], trans_a, sem).wait()

        def wait_in(k):
            pltpu.make_async_copy(
                tt_hbm.at[:, pl.ds(0, _CT)], chunks_v.at[k],
                sems_in.at[k]).wait()

        ring = ((trans_a, sem_a), (trans_b, sem_b),
                (trans_c, sem_c), (trans_d, sem_d))

        def chunk_body(k4, _):
            k = k4 * 4
            for s, (trans, sem) in enumerate(ring):
                pl.when(k4 > 0)(lambda sem=sem: drain(sem))
                wait_in(k + s)
                transpose_chunk(chunks_v.at[k + s], trans)
                out_copy(trans, (base_ct + k + s) * _CT * _D, _CT * _D, sem)
            return 0

        lax.fori_loop(0, _K_PER_W // 4, chunk_body, 0, unroll=False)
        for _, sem in ring:
            drain(sem)
        trans, sem = trans_a, sem_a

        @pl.when(wid < _N_EXTRA)
        def _():
            hx.wait()
            transpose_chunk(chunk_x, trans)
            out_copy(trans, (_EXTRA0 + wid) * _CT * _D, _CT * _D, sem).wait()

        @pl.when(wid == _N_EXTRA)
        def _():
            ht.wait()
            transpose_chunk(chunk_t, trans)
            out_copy(trans, _NFULL_TILES * _CT * _D, _TAIL * _D, sem).wait()

    return detile_kernel


def _make_gather(batch: int, dim: int):
    info = plsc.get_sparse_core_info()
    num_workers = info.num_cores * info.num_subcores
    b_per_w = batch // num_workers
    mesh = plsc.VectorSubcoreMesh(core_axis_name="c", subcore_axis_name="s")

    @functools.partial(
        pl.kernel,
        mesh=mesh,
        out_type=jax.ShapeDtypeStruct((dim, batch), jnp.float32),
        scratch_types=[
            pltpu.VMEM((b_per_w,), jnp.int32),
            pltpu.VMEM((b_per_w, dim), jnp.float32),
            pltpu.VMEM((dim, b_per_w), jnp.float32),
            pltpu.SemaphoreType.DMA,
        ],
        compiler_params=pltpu.CompilerParams(
            use_tc_tiling_on_sc=False, needs_layout_passes=False),
    )
    def gather_kernel(table_hbm, idx_hbm, out_hbm, idx_v, rows_v, panel_v,
                      sem):
        wid = lax.axis_index("s") * info.num_cores + lax.axis_index("c")
        base = wid * b_per_w
        pltpu.sync_copy(idx_hbm.at[pl.ds(base, b_per_w)], idx_v)
        pltpu.async_copy(table_hbm.at[idx_v], rows_v, sem).wait()

        iota = lax.iota(jnp.int32, _LANES)
        rot = [(iota + r) & (_LANES - 1) for r in range(_LANES)]

        def body(g, _):
            bj = g * _LANES
            for bd in (0, _LANES):
                dvec = iota + bd
                for r in range(_LANES):
                    jvec = rot[r] + bj
                    v = plsc.load_gather(rows_v, [jvec, dvec])
                    plsc.store_scatter(panel_v, [dvec, jvec], v)
            return 0
        lax.fori_loop(0, b_per_w // _LANES, body, 0, unroll=False)

        pltpu.sync_copy(panel_v, out_hbm.at[:, pl.ds(base, b_per_w)])

    return gather_kernel


@functools.lru_cache(maxsize=None)
def _pipeline(batch, dim):
    detile = _make_detile()
    gather = _make_gather(batch, dim)

    def run(indices, table):
        tlin = detile(table.T).reshape(_V, _D)
        return gather(tlin, indices.astype(jnp.int32)).T

    return run


def kernel(indices, table):
    batch, = indices.shape
    _, dim = table.shape
    return _pipeline(batch, dim)(indices, table)


# speedup vs baseline: 1.7453x; 1.0014x over previous
import functools

import jax
import jax.numpy as jnp
from jax import lax
from jax.experimental import pallas as pl
from jax.experimental.pallas import tpu as pltpu
from jax.experimental.pallas import tpu_sc as plsc

_V = 100001
_D = 32
_LANES = 16
_CT = 128
_NFULL_TILES = _V // _CT
_TAIL = _V - _NFULL_TILES * _CT
_NW = 32
_K_PER_W = _NFULL_TILES // _NW
_EXTRA0 = _NW * _K_PER_W
_N_EXTRA = _NFULL_TILES - _EXTRA0
_TAIL_GRP = 3


def _make_detile():
    mesh = plsc.VectorSubcoreMesh(core_axis_name="c", subcore_axis_name="s")

    @functools.partial(
        pl.kernel,
        mesh=mesh,
        out_type=jax.ShapeDtypeStruct((_V * _D,), jnp.float32),
        scratch_types=[
            pltpu.VMEM((_K_PER_W, _D, _CT), jnp.float32),
            pltpu.VMEM((_D, _CT), jnp.float32),
            pltpu.VMEM((_D, _CT), jnp.float32),
            pltpu.VMEM((_CT * _D,), jnp.float32),
            pltpu.VMEM((_CT * _D,), jnp.float32),
            pltpu.VMEM((_CT * _D,), jnp.float32),
            pltpu.VMEM((_CT * _D,), jnp.float32),
            pltpu.SemaphoreType.DMA((_K_PER_W,)),
            pltpu.SemaphoreType.DMA,
            pltpu.SemaphoreType.DMA,
            pltpu.SemaphoreType.DMA,
            pltpu.SemaphoreType.DMA,
            pltpu.SemaphoreType.DMA,
        ],
        compiler_params=pltpu.CompilerParams(
            use_tc_tiling_on_sc=True, needs_layout_passes=False,
            disable_bounds_checks=True),
    )
    def detile_kernel(tt_hbm, out_hbm, chunks_v, chunk_x, chunk_t,
                      trans_a, trans_b, trans_c, trans_d, sems_in, sem_x,
                      sem_a, sem_b, sem_c, sem_d):
        wid = lax.axis_index("s") * 2 + lax.axis_index("c")
        base_ct = wid * _K_PER_W

        def fire_in(k):
            pltpu.make_async_copy(
                tt_hbm.at[:, pl.ds((base_ct + k) * _CT, _CT)],
                chunks_v.at[k], sems_in.at[k]).start()

        for k in range(8):
            fire_in(k)
        hx = pltpu.make_async_copy(
            tt_hbm.at[:, pl.ds((_EXTRA0 + wid) * _CT, _CT)], chunk_x, sem_x)
        tail_off = pl.multiple_of(
            jnp.int32(_NFULL_TILES * _CT) + wid * 0, _CT)
        ht = pltpu.make_async_copy(
            tt_hbm.at[:, pl.ds(tail_off, _CT)], chunk_t, sem_x)
        pl.when(wid < _N_EXTRA)(hx.start)
        pl.when(wid == _N_EXTRA)(ht.start)

        iota = lax.iota(jnp.int32, _LANES)
        ngrp = _CT // _LANES
        rot = [(iota + r) & (_LANES - 1) for r in range(_LANES)]
        trix = [rot[r] * _D + iota for r in range(_LANES)]

        def transpose_chunk(src, trans):
            def body(g, _):
                bj = g * _LANES
                for bd in (0, _LANES):
                    dvec = iota + bd
                    base_tr = bj * _D + bd
                    for r in range(_LANES):
                        v = plsc.load_gather(src, [dvec, rot[r] + bj])
                        plsc.store_scatter(trans, [trix[r] + base_tr], v)
                return 0
            lax.fori_loop(0, ngrp, body, 0, unroll=False)

        def out_copy(trans, off, n, sem):
            h = pltpu.make_async_copy(
                trans.at[pl.ds(0, n)], out_hbm.at[pl.ds(off, n)], sem)
            h.start()
            return h

        def drain(sem):
            pltpu.make_async_copy(
                out_hbm.at[pl.ds(0, _CT * _D)], trans_a, sem).wait()

        def wait_in(k):
            pltpu.make_async_copy(
                tt_hbm.at[:, pl.ds(0, _CT)], chunks_v.at[k],
                sems_in.at[k]).wait()

        ring = ((trans_a, sem_a), (trans_b, sem_b),
                (trans_c, sem_c), (trans_d, sem_d))

        def chunk_body(k4, _):
            k = k4 * 4
            for s, (trans, sem) in enumerate(ring):
                pl.when(k4 < (_K_PER_W - 8) // 4)(
                    lambda k=k, s=s: fire_in(k + 8 + s))
                pl.when(k4 > 0)(lambda sem=sem: drain(sem))
                wait_in(k + s)
                transpose_chunk(chunks_v.at[k + s], trans)
                out_copy(trans, (base_ct + k + s) * _CT * _D, _CT * _D, sem)
            return 0

        lax.fori_loop(0, _K_PER_W // 4, chunk_body, 0, unroll=False)
        for _, sem in ring:
            drain(sem)
        trans, sem = trans_a, sem_a

        @pl.when(wid < _N_EXTRA)
        def _():
            hx.wait()
            transpose_chunk(chunk_x, trans)
            out_copy(trans, (_EXTRA0 + wid) * _CT * _D, _CT * _D, sem).wait()

        @pl.when(wid == _N_EXTRA)
        def _():
            ht.wait()
            transpose_chunk(chunk_t, trans)
            out_copy(trans, _NFULL_TILES * _CT * _D, _TAIL * _D, sem).wait()

    return detile_kernel


def _make_gather(batch: int, dim: int):
    info = plsc.get_sparse_core_info()
    num_workers = info.num_cores * info.num_subcores
    b_per_w = batch // num_workers
    mesh = plsc.VectorSubcoreMesh(core_axis_name="c", subcore_axis_name="s")

    @functools.partial(
        pl.kernel,
        mesh=mesh,
        out_type=jax.ShapeDtypeStruct((dim, batch), jnp.float32),
        scratch_types=[
            pltpu.VMEM((b_per_w,), jnp.int32),
            pltpu.VMEM((b_per_w, dim), jnp.float32),
            pltpu.VMEM((dim, b_per_w), jnp.float32),
            pltpu.SemaphoreType.DMA,
        ],
        compiler_params=pltpu.CompilerParams(
            use_tc_tiling_on_sc=False, needs_layout_passes=False),
    )
    def gather_kernel(table_hbm, idx_hbm, out_hbm, idx_v, rows_v, panel_v,
                      sem):
        wid = lax.axis_index("s") * info.num_cores + lax.axis_index("c")
        base = wid * b_per_w
        pltpu.sync_copy(idx_hbm.at[pl.ds(base, b_per_w)], idx_v)
        pltpu.async_copy(table_hbm.at[idx_v], rows_v, sem).wait()

        iota = lax.iota(jnp.int32, _LANES)
        rot = [(iota + r) & (_LANES - 1) for r in range(_LANES)]

        def body(g, _):
            bj = g * _LANES
            for bd in (0, _LANES):
                dvec = iota + bd
                for r in range(_LANES):
                    jvec = rot[r] + bj
                    v = plsc.load_gather(rows_v, [jvec, dvec])
                    plsc.store_scatter(panel_v, [dvec, jvec], v)
            return 0
        lax.fori_loop(0, b_per_w // _LANES, body, 0, unroll=False)

        pltpu.sync_copy(panel_v, out_hbm.at[:, pl.ds(base, b_per_w)])

    return gather_kernel


@functools.lru_cache(maxsize=None)
def _pipeline(batch, dim):
    detile = _make_detile()
    gather = _make_gather(batch, dim)

    def run(indices, table):
        tlin = detile(table.T).reshape(_V, _D)
        return gather(tlin, indices.astype(jnp.int32)).T

    return run


def kernel(indices, table):
    batch, = indices.shape
    _, dim = table.shape
    return _pipeline(batch, dim)(indices, table)
